# baseline (speedup 1.0000x reference)
import functools

import jax
import jax.numpy as jnp
from jax import lax
from jax.experimental import pallas as pl
from jax.experimental.pallas import tpu as pltpu
from jax.experimental.pallas import tpu_sc as plsc

B, L = 1024, 200
N = B * L
W = 100001
NB1 = 8
NB2 = 8
D = NB1 + NB2
MROWS = 12 * NB1
WROWS = 52 * NB2

NC, NS = 2, 16
NWK = NC * NS
PER_W = N // NWK
C = 1600
CHUNKS = PER_W // C
VC = C // 2


def _body(item_hbm, t1_hbm, t2_hbm, month_hbm, week_hbm, out_hbm,
          item_v, t1_v, t2_v, midx_v, widx_v, m_v, w_v, o_v, sem_m, sem_w):
    iota = lax.iota(jnp.int32, 16)
    psel = lax.shift_right_logical(iota, 3)
    off = (iota & 7) * W
    rot8 = (iota + 8) & 15
    lt8 = iota < 8

    wid = lax.axis_index("s") * NC + lax.axis_index("c")

    for ci in range(CHUNKS):
        base = wid * PER_W + ci * C
        pltpu.sync_copy(item_hbm.at[pl.ds(base, C)], item_v)
        pltpu.sync_copy(t1_hbm.at[pl.ds(base, C)], t1_v)
        pltpu.sync_copy(t2_hbm.at[pl.ds(base, C)], t2_v)

        def build(g, _):
            it16 = item_v[pl.ds(16 * g, 16)]
            mb16 = t1_v[pl.ds(16 * g, 16)] * (NB1 * W) + it16
            wb16 = t2_v[pl.ds(16 * g, 16)] * (NB2 * W) + it16
            for k in range(8):
                vb = 16 * (8 * g + k)
                midx_v[pl.ds(vb, 16)] = jnp.where(lt8, mb16[2 * k], mb16[2 * k + 1]) + off
                widx_v[pl.ds(vb, 16)] = jnp.where(lt8, wb16[2 * k], wb16[2 * k + 1]) + off
            return 0

        lax.fori_loop(0, C // 16, build, 0)

        cpm = pltpu.make_async_copy(month_hbm.at[midx_v], m_v, sem_m)
        cpw = pltpu.make_async_copy(week_hbm.at[widx_v], w_v, sem_w)
        cpm.start()
        cpw.start()
        cpm.wait()
        cpw.wait()

        def merge(v, _):
            mv = m_v[pl.ds(16 * v, 16)]
            wv = w_v[pl.ds(16 * v, 16)]
            mrot = mv.at[rot8].get(mode="promise_in_bounds")
            wrot = wv.at[rot8].get(mode="promise_in_bounds")
            o_v[pl.ds(32 * v, 16)] = jnp.where(lt8, mv, wrot)
            o_v[pl.ds(32 * v + 16, 16)] = jnp.where(lt8, mrot, wv)
            return 0

        lax.fori_loop(0, VC, merge, 0)

        pltpu.sync_copy(o_v, out_hbm.at[pl.ds(D * base, D * C)])


@jax.jit
def _popularity_encode(item_flat, t1_flat, t2_flat, month_flat, week_flat):
    mesh = plsc.VectorSubcoreMesh(core_axis_name="c", subcore_axis_name="s")
    run = pl.kernel(
        _body,
        out_type=jax.ShapeDtypeStruct((N * D,), jnp.float32),
        mesh=mesh,
        scratch_types=[
            pltpu.VMEM((C,), jnp.int32),
            pltpu.VMEM((C,), jnp.int32),
            pltpu.VMEM((C,), jnp.int32),
            pltpu.VMEM((C * NB1,), jnp.int32),
            pltpu.VMEM((C * NB2,), jnp.int32),
            pltpu.VMEM((C * NB1,), jnp.float32),
            pltpu.VMEM((C * NB2,), jnp.float32),
            pltpu.VMEM((C * D,), jnp.float32),
            pltpu.SemaphoreType.DMA,
            pltpu.SemaphoreType.DMA,
        ],
        name="popularity_encoding_sc",
    )
    return run(item_flat, t1_flat, t2_flat, month_flat, week_flat)


def kernel(log_seqs, time1_seqs, time2_seqs, month_pop_table, week_pop_table):
    item_flat = log_seqs.reshape(-1).astype(jnp.int32)
    t1_flat = time1_seqs.reshape(-1).astype(jnp.int32)
    t2_flat = time2_seqs.reshape(-1).astype(jnp.int32)
    month_flat = month_pop_table.reshape(-1)
    week_flat = week_pop_table.reshape(-1)
    out = _popularity_encode(item_flat, t1_flat, t2_flat, month_flat, week_flat)
    return out.reshape(B, L, D)
